# baseline (speedup 1.0000x reference)
import functools

import jax
import jax.numpy as jnp
from jax import lax
from jax.experimental import pallas as pl
from jax.experimental.pallas import tpu as pltpu
from jax.experimental.pallas import tpu_sc as plsc

B = 16384
D = 64
NC = 2
NS = 16
NW = NC * NS
BPW = B // NW
K = 16
NCHUNK = BPW // K
VEC = 16

_mesh = plsc.VectorSubcoreMesh(core_axis_name="c", subcore_axis_name="s")


@functools.partial(
    pl.kernel,
    mesh=_mesh,
    out_type=jax.ShapeDtypeStruct((B, 2 * D), jnp.float32),
    scratch_types=[
        pltpu.VMEM((BPW,), jnp.int32),
        pltpu.VMEM((BPW,), jnp.int32),
        pltpu.SMEM((BPW,), jnp.int32),
        pltpu.SMEM((BPW,), jnp.int32),
        pltpu.VMEM((K * 8, D), jnp.float32),
        pltpu.VMEM((K * 8, D), jnp.float32),
        pltpu.VMEM((K * 8, D), jnp.float32),
        pltpu.VMEM((K * 8, D), jnp.float32),
        pltpu.VMEM((K, 2 * D), jnp.float32),
        pltpu.VMEM((K, 2 * D), jnp.float32),
        pltpu.SemaphoreType.DMA,
        pltpu.SemaphoreType.DMA,
        pltpu.SemaphoreType.DMA,
        pltpu.SemaphoreType.DMA,
    ],
)
def _emb_kernel(uid_hbm, iid_hbm, tuid_hbm, tiid_hbm, out_hbm,
                uidx_v, iidx_v, us_s, is_s,
                ugrp_a, ugrp_b, igrp_a, igrp_b, crows_a, crows_b,
                sem_a, sem_b, sem_wa, sem_wb):
    wid = lax.axis_index("s") * NC + lax.axis_index("c")
    base = wid * BPW

    pltpu.sync_copy(uid_hbm.at[pl.ds(base, BPW)], uidx_v)
    pltpu.sync_copy(iid_hbm.at[pl.ds(base, BPW)], iidx_v)

    def _stage(t, carry):
        xu = uidx_v[pl.ds(t * VEC, VEC)]
        xi = iidx_v[pl.ds(t * VEC, VEC)]
        for l in range(VEC):
            us_s[t * VEC + l] = xu[l]
            is_s[t * VEC + l] = xi[l]
        return carry

    lax.fori_loop(0, BPW // VEC, _stage, 0)

    def _fire(c, ugrp, igrp, sem):
        cbase = c * K

        def body(j, carry):
            gu = pl.multiple_of((us_s[cbase + j] >> 3) << 3, 8)
            gi = pl.multiple_of((is_s[cbase + j] >> 3) << 3, 8)
            pltpu.async_copy(tuid_hbm.at[pl.ds(gu, 8)],
                             ugrp.at[pl.ds(j * 8, 8)], sem)
            pltpu.async_copy(tiid_hbm.at[pl.ds(gi, 8)],
                             igrp.at[pl.ds(j * 8, 8)], sem)
            return carry

        lax.fori_loop(0, K, body, 0)

    def _drain(ugrp, igrp, sem):
        pltpu.make_async_copy(tuid_hbm.at[pl.ds(0, K * 8)], ugrp,
                              sem).wait()
        pltpu.make_async_copy(tiid_hbm.at[pl.ds(0, K * 8)], igrp,
                              sem).wait()

    def _extract(c, ugrp, igrp, crows):
        cbase = c * K

        def body(j, carry):
            su = j * 8 + (us_s[cbase + j] & 7)
            si = j * 8 + (is_s[cbase + j] & 7)
            for k in range(D // VEC):
                crows[j, pl.ds(k * VEC, VEC)] = (
                    ugrp[su, pl.ds(k * VEC, VEC)])
                crows[j, pl.ds(D + k * VEC, VEC)] = (
                    igrp[si, pl.ds(k * VEC, VEC)])
            return carry

        lax.fori_loop(0, K, body, 0)

    def _wdesc(crows, sem_w, c):
        off = pl.multiple_of(base + c * K, 8)
        return pltpu.make_async_copy(
            crows, out_hbm.at[pl.ds(off, K)], sem_w)

    _fire(0, ugrp_a, igrp_a, sem_a)

    def _pair(c, carry):
        e = 2 * c

        @pl.when(c > 0)
        def _():
            _wdesc(crows_a, sem_wa, 0).wait()

        _fire(e + 1, ugrp_b, igrp_b, sem_b)
        _drain(ugrp_a, igrp_a, sem_a)
        _extract(e, ugrp_a, igrp_a, crows_a)
        _wdesc(crows_a, sem_wa, e).start()

        @pl.when(c > 0)
        def _():
            _wdesc(crows_b, sem_wb, 0).wait()

        @pl.when(c + 1 < NCHUNK // 2)
        def _():
            _fire(e + 2, ugrp_a, igrp_a, sem_a)

        _drain(ugrp_b, igrp_b, sem_b)
        _extract(e + 1, ugrp_b, igrp_b, crows_b)
        _wdesc(crows_b, sem_wb, e + 1).start()
        return carry

    lax.fori_loop(0, NCHUNK // 2, _pair, 0)
    _wdesc(crows_a, sem_wa, 0).wait()
    _wdesc(crows_b, sem_wb, 0).wait()


def kernel(uid, iid, table_uid, table_iid):
    uid = uid.astype(jnp.int32)
    iid = iid.astype(jnp.int32)
    decoy = jnp.take(table_uid, uid[:1], axis=0)
    uid = uid + (decoy[0, 0] * 0.0).astype(jnp.int32)
    return _emb_kernel(uid, iid, table_uid, table_iid)


# speedup vs baseline: 1.0194x; 1.0194x over previous
import functools

import jax
import jax.numpy as jnp
from jax import lax
from jax.experimental import pallas as pl
from jax.experimental.pallas import tpu as pltpu
from jax.experimental.pallas import tpu_sc as plsc

B = 16384
D = 64
NC = 2
NS = 16
NW = NC * NS
BPW = B // NW
K = 16
NCHUNK = BPW // K
VEC = 16

_mesh = plsc.VectorSubcoreMesh(core_axis_name="c", subcore_axis_name="s")


@functools.partial(
    pl.kernel,
    mesh=_mesh,
    out_type=jax.ShapeDtypeStruct((B, 2 * D), jnp.float32),
    scratch_types=[
        pltpu.VMEM((BPW,), jnp.int32),
        pltpu.VMEM((BPW,), jnp.int32),
        pltpu.SMEM((BPW,), jnp.int32),
        pltpu.SMEM((BPW,), jnp.int32),
        pltpu.VMEM((K * 8, D), jnp.float32),
        pltpu.VMEM((K * 8, D), jnp.float32),
        pltpu.VMEM((K * 8, D), jnp.float32),
        pltpu.VMEM((K * 8, D), jnp.float32),
        pltpu.VMEM((K, 2 * D), jnp.float32),
        pltpu.VMEM((K, 2 * D), jnp.float32),
        pltpu.SemaphoreType.DMA,
        pltpu.SemaphoreType.DMA,
        pltpu.SemaphoreType.DMA,
        pltpu.SemaphoreType.DMA,
    ],
)
def _emb_kernel(uid_hbm, iid_hbm, tuid_hbm, tiid_hbm, out_hbm,
                uidx_v, iidx_v, us_s, is_s,
                ugrp_a, ugrp_b, igrp_a, igrp_b, crows_a, crows_b,
                sem_a, sem_b, sem_wa, sem_wb):
    wid = lax.axis_index("s") * NC + lax.axis_index("c")
    base = wid * BPW

    pltpu.sync_copy(uid_hbm.at[pl.ds(base, BPW)], uidx_v)
    pltpu.sync_copy(iid_hbm.at[pl.ds(base, BPW)], iidx_v)

    def _stage(t, carry):
        xu = uidx_v[pl.ds(t * VEC, VEC)]
        xi = iidx_v[pl.ds(t * VEC, VEC)]
        for l in range(VEC):
            us_s[t * VEC + l] = xu[l]
            is_s[t * VEC + l] = xi[l]
        return carry

    lax.fori_loop(0, BPW // VEC, _stage, 0)

    def _fire(c, ugrp, igrp, sem):
        cbase = c * K

        def body(j, carry):
            gu = pl.multiple_of((us_s[cbase + j] >> 3) << 3, 8)
            gi = pl.multiple_of((is_s[cbase + j] >> 3) << 3, 8)
            pltpu.async_copy(tuid_hbm.at[pl.ds(gu, 8)],
                             ugrp.at[pl.ds(j * 8, 8)], sem)
            pltpu.async_copy(tiid_hbm.at[pl.ds(gi, 8)],
                             igrp.at[pl.ds(j * 8, 8)], sem)
            return carry

        lax.fori_loop(0, K, body, 0)

    def _drain(ugrp, igrp, sem):
        pltpu.make_async_copy(tuid_hbm.at[pl.ds(0, K * 8)], ugrp,
                              sem).wait()
        pltpu.make_async_copy(tiid_hbm.at[pl.ds(0, K * 8)], igrp,
                              sem).wait()

    def _extract(c, ugrp, igrp, crows):
        cbase = c * K

        def body(j, carry):
            su = j * 8 + (us_s[cbase + j] & 7)
            si = j * 8 + (is_s[cbase + j] & 7)
            for k in range(D // VEC):
                crows[j, pl.ds(k * VEC, VEC)] = (
                    ugrp[su, pl.ds(k * VEC, VEC)])
                crows[j, pl.ds(D + k * VEC, VEC)] = (
                    igrp[si, pl.ds(k * VEC, VEC)])
            return carry

        lax.fori_loop(0, K, body, 0)

    def _wdesc(crows, sem_w, c):
        off = pl.multiple_of(base + c * K, 8)
        return pltpu.make_async_copy(
            crows, out_hbm.at[pl.ds(off, K)], sem_w)

    _fire(0, ugrp_a, igrp_a, sem_a)

    def _pair(c, carry):
        e = 2 * c

        @pl.when(c > 0)
        def _():
            _wdesc(crows_a, sem_wa, 0).wait()

        _fire(e + 1, ugrp_b, igrp_b, sem_b)
        _drain(ugrp_a, igrp_a, sem_a)
        _extract(e, ugrp_a, igrp_a, crows_a)
        _wdesc(crows_a, sem_wa, e).start()

        @pl.when(c > 0)
        def _():
            _wdesc(crows_b, sem_wb, 0).wait()

        @pl.when(c + 1 < NCHUNK // 2)
        def _():
            _fire(e + 2, ugrp_a, igrp_a, sem_a)

        _drain(ugrp_b, igrp_b, sem_b)
        _extract(e + 1, ugrp_b, igrp_b, crows_b)
        _wdesc(crows_b, sem_wb, e + 1).start()
        return carry

    lax.fori_loop(0, NCHUNK // 2, _pair, 0)
    _wdesc(crows_a, sem_wa, 0).wait()
    _wdesc(crows_b, sem_wb, 0).wait()


def kernel(uid, iid, table_uid, table_iid):
    uid = uid.astype(jnp.int32)
    iid = iid.astype(jnp.int32)
    return _emb_kernel(uid, iid, table_uid, table_iid)
